# baseline (speedup 1.0000x reference)
import jax
import jax.numpy as jnp
from jax.experimental import pallas as pl

B_BLK = 128


def _kern(ids_ref, emb_ref, pe_ref, ue_ref, mask_ref):
    ids = ids_ref[...]
    mask = (ids != 0).astype(jnp.float32)
    emb = emb_ref[...]
    pe = pe_ref[...]
    m3 = mask[:, :, None]
    scale = float(emb.shape[-1]) ** 0.5
    ue_ref[...] = (emb * scale + pe[None]) * m3
    mask_ref[...] = mask


def kernel(past_lengths, past_ids, past_embeddings, past_payloads, pos_emb):
    B, N = past_ids.shape
    D = past_embeddings.shape[-1]
    grid = (B // B_BLK,)
    ue, mask = pl.pallas_call(
        _kern,
        grid=grid,
        in_specs=[
            pl.BlockSpec((B_BLK, N), lambda i: (i, 0)),
            pl.BlockSpec((B_BLK, N, D), lambda i: (i, 0, 0)),
            pl.BlockSpec((N, D), lambda i: (0, 0)),
        ],
        out_specs=[
            pl.BlockSpec((B_BLK, N, D), lambda i: (i, 0, 0)),
            pl.BlockSpec((B_BLK, N), lambda i: (i, 0)),
        ],
        out_shape=[
            jax.ShapeDtypeStruct((B, N, D), jnp.float32),
            jax.ShapeDtypeStruct((B, N), jnp.float32),
        ],
    )(past_ids, past_embeddings, pos_emb)
    return (past_lengths, ue, mask[..., None])


# speedup vs baseline: 1.4935x; 1.4935x over previous
import jax
import jax.numpy as jnp
from jax.experimental import pallas as pl
from jax.experimental.pallas import tpu as pltpu

B_BLK = 128


def _kern(ids_ref, emb_ref, pe_ref, ue_ref, mask_ref):
    ids = ids_ref[...]
    mask = (ids != 0).astype(jnp.float32)
    mask_ref[...] = mask
    N = ids.shape[1]
    D = emb_ref.shape[1] // N
    scale = float(D) ** 0.5
    mask_rep = jnp.broadcast_to(mask[:, :, None], (mask.shape[0], N, D)).reshape(
        mask.shape[0], N * D
    )
    ue_ref[...] = (emb_ref[...] * scale + pe_ref[...]) * mask_rep


def kernel(past_lengths, past_ids, past_embeddings, past_payloads, pos_emb):
    B, N = past_ids.shape
    D = past_embeddings.shape[-1]
    emb2 = past_embeddings.reshape(B, N * D)
    pe2 = pos_emb.reshape(1, N * D)
    grid = (B // B_BLK,)
    ue, mask = pl.pallas_call(
        _kern,
        grid=grid,
        in_specs=[
            pl.BlockSpec((B_BLK, N), lambda i: (i, 0)),
            pl.BlockSpec((B_BLK, N * D), lambda i: (i, 0)),
            pl.BlockSpec((1, N * D), lambda i: (0, 0)),
        ],
        out_specs=[
            pl.BlockSpec((B_BLK, N * D), lambda i: (i, 0)),
            pl.BlockSpec((B_BLK, N), lambda i: (i, 0)),
        ],
        out_shape=[
            jax.ShapeDtypeStruct((B, N * D), jnp.float32),
            jax.ShapeDtypeStruct((B, N), jnp.float32),
        ],
        compiler_params=pltpu.CompilerParams(
            dimension_semantics=("parallel",),
        ),
    )(past_ids, emb2, pe2)
    return (past_lengths, ue.reshape(B, N, D), mask[..., None])
